# baseline (speedup 1.0000x reference)
import functools
import math

import jax
import jax.numpy as jnp
from jax import lax
from jax.experimental import pallas as pl
from jax.experimental.pallas import tpu as pltpu
from jax.experimental.pallas import tpu_sc as plsc

VOCAB = 1000000
D = 64
BATCH = 16384
HIST = 200

NC, NS, L = 2, 16, 16
NW = NC * NS

BB = 128
NBB = BATCH // BB
NCHUNK = HIST * NBB
NB = NCHUNK // NW
NBUF = 4
NT = NB // NBUF

SCALE = math.sqrt(D)

ROWS_PER_ITER = 8


def _make_sc_kernel():
  mesh = plsc.VectorSubcoreMesh(
      core_axis_name="c", subcore_axis_name="s", num_cores=NC
  )

  scratch = (
      [pltpu.VMEM((NBUF, BB), jnp.int32)]
      + [pltpu.VMEM((BB, D), jnp.float32)] * NBUF
      + [pltpu.VMEM((D * BB,), jnp.float32)] * NBUF
      + [pltpu.SemaphoreType.DMA] * (3 * NBUF)
  )

  @functools.partial(
      pl.kernel,
      mesh=mesh,
      out_type=jax.ShapeDtypeStruct((HIST, D // 8, NBB, 8 * BB), jnp.float32),
      compiler_params=pltpu.CompilerParams(
          use_tc_tiling_on_sc=False, needs_layout_passes=False
      ),
      scratch_types=scratch,
  )
  def sc_kernel(idx_hbm, table_hbm, out_hbm, idx4, *rest):
    rows = rest[0:NBUF]
    tbuf = rest[NBUF : 2 * NBUF]
    sems = rest[2 * NBUF :]
    gsem = sems[0:NBUF]
    wsem = sems[NBUF : 2 * NBUF]
    isem = sems[2 * NBUF : 3 * NBUF]

    wid = lax.axis_index("s") * NC + lax.axis_index("c")
    wblk = wid * NB

    def chunk_hb(g):
      c = wblk + g
      return c // NBB, c % NBB

    def idx_sync(g, b):
      h, bb = chunk_hb(g)
      pltpu.sync_copy(idx_hbm.at[h, pl.ds(bb * BB, BB)], idx4.at[b])

    def idx_fire(g, b):
      h, bb = chunk_hb(g)
      pltpu.async_copy(idx_hbm.at[h, pl.ds(bb * BB, BB)], idx4.at[b], isem[b])

    def idx_wait(g, b):
      h, bb = chunk_hb(g)
      pltpu.make_async_copy(
          idx_hbm.at[h, pl.ds(bb * BB, BB)], idx4.at[b], isem[b]
      ).wait()

    def gather_fire(b):
      pltpu.async_copy(table_hbm.at[idx4.at[b]], rows[b], gsem[b])

    def gather_wait(b):
      pltpu.make_async_copy(table_hbm.at[idx4.at[b]], rows[b], gsem[b]).wait()

    def wb_fire(g, b):
      h, bb = chunk_hb(g)
      for dhi in range(D // 8):
        pltpu.async_copy(
            tbuf[b].at[pl.ds(dhi * 8 * BB, 8 * BB)],
            out_hbm.at[h, dhi, bb],
            wsem[b],
        )

    def wb_wait(g, b):
      h, bb = chunk_hb(g)
      for dhi in range(D // 8):
        pltpu.make_async_copy(
            tbuf[b].at[pl.ds(dhi * 8 * BB, 8 * BB)],
            out_hbm.at[h, dhi, bb],
            wsem[b],
        ).wait()

    dvec = lax.iota(jnp.int32, L)
    doff = [
        (jnp.int32(2 * q) + (dvec >> 3)) * (8 * BB) + (dvec & 7) * BB
        for q in range(D // L)
    ]

    def transpose_scale(b):
      def body(r, c):
        base = r * ROWS_PER_ITER
        for i in range(ROWS_PER_ITER):
          bvec = jnp.full((L,), base + i, jnp.int32)
          for q in range(D // L):
            v = rows[b][base + i, pl.ds(q * L, L)] * SCALE
            plsc.store_scatter(tbuf[b], [doff[q] + bvec], v)
        return c

      lax.fori_loop(0, BB // ROWS_PER_ITER, body, 0)

    idx_sync(0, 0)
    idx_sync(1, 1)
    idx_sync(2, 2)
    gather_fire(0)
    gather_fire(1)

    def step(g, k, *, skip_isem_wait=False, fire_idx=True, refill=True,
             wait_wb=True):
      gather_wait(k)
      if wait_wb:
        wb_wait(g - NBUF, k)
      transpose_scale(k)
      wb_fire(g, k)
      if refill:
        b2 = (k + 2) % NBUF
        if not skip_isem_wait:
          idx_wait(g + 2, b2)
        gather_fire(b2)
        if fire_idx:
          idx_fire(g + 3, (k + 3) % NBUF)

    step(0, 0, skip_isem_wait=True, wait_wb=False)
    step(1, 1, wait_wb=False)
    step(2, 2, wait_wb=False)
    step(3, 3, wait_wb=False)

    def group(t, c):
      for k in range(NBUF):
        step(t * NBUF + k, k)
      return c

    lax.fori_loop(1, NT - 1, group, 0)

    g0 = NB - NBUF
    step(g0 + 0, 0)
    step(g0 + 1, 1, fire_idx=False)
    step(g0 + 2, 2, refill=False)
    step(g0 + 3, 3, refill=False)

    for k in range(NBUF):
      wb_wait(g0 + k, k)

  return sc_kernel


def kernel(x, table):
  out4 = _make_sc_kernel()(x.T.astype(jnp.int32), table)
  out5 = out4.reshape(HIST, D // 8, NBB, 8, BB)
  return out5.transpose(2, 4, 0, 1, 3).reshape(BATCH, HIST, D)


# speedup vs baseline: 1.0065x; 1.0065x over previous
import functools
import math

import jax
import jax.numpy as jnp
from jax import lax
from jax.experimental import pallas as pl
from jax.experimental.pallas import tpu as pltpu
from jax.experimental.pallas import tpu_sc as plsc

VOCAB = 1000000
D = 64
BATCH = 16384
HIST = 200

NC, NS, L = 2, 16, 16
NW = NC * NS

BB = 128
NBB = BATCH // BB
NCHUNK = HIST * NBB
NB = NCHUNK // NW
NBUF = 4
NT = NB // NBUF

SCALE = math.sqrt(D)

ROWS_PER_ITER = 16


def _make_sc_kernel():
  mesh = plsc.VectorSubcoreMesh(
      core_axis_name="c", subcore_axis_name="s", num_cores=NC
  )

  scratch = (
      [pltpu.VMEM((NBUF, BB), jnp.int32)]
      + [pltpu.VMEM((BB, D), jnp.float32)] * NBUF
      + [pltpu.VMEM((D * BB,), jnp.float32)] * NBUF
      + [pltpu.SemaphoreType.DMA] * (3 * NBUF)
  )

  @functools.partial(
      pl.kernel,
      mesh=mesh,
      out_type=jax.ShapeDtypeStruct((HIST, D // 8, NBB, 8 * BB), jnp.float32),
      compiler_params=pltpu.CompilerParams(
          use_tc_tiling_on_sc=False, needs_layout_passes=False
      ),
      scratch_types=scratch,
  )
  def sc_kernel(idx_hbm, table_hbm, out_hbm, idx4, *rest):
    rows = rest[0:NBUF]
    tbuf = rest[NBUF : 2 * NBUF]
    sems = rest[2 * NBUF :]
    gsem = sems[0:NBUF]
    wsem = sems[NBUF : 2 * NBUF]
    isem = sems[2 * NBUF : 3 * NBUF]

    wid = lax.axis_index("s") * NC + lax.axis_index("c")
    wblk = wid * NB

    def chunk_hb(g):
      c = wblk + g
      return c // NBB, c % NBB

    def idx_sync(g, b):
      h, bb = chunk_hb(g)
      pltpu.sync_copy(idx_hbm.at[h, pl.ds(bb * BB, BB)], idx4.at[b])

    def idx_fire(g, b):
      h, bb = chunk_hb(g)
      pltpu.async_copy(idx_hbm.at[h, pl.ds(bb * BB, BB)], idx4.at[b], isem[b])

    def idx_wait(g, b):
      h, bb = chunk_hb(g)
      pltpu.make_async_copy(
          idx_hbm.at[h, pl.ds(bb * BB, BB)], idx4.at[b], isem[b]
      ).wait()

    def gather_fire(b):
      pltpu.async_copy(table_hbm.at[idx4.at[b]], rows[b], gsem[b])

    def gather_wait(b):
      pltpu.make_async_copy(table_hbm.at[idx4.at[b]], rows[b], gsem[b]).wait()

    def wb_fire(g, b):
      h, bb = chunk_hb(g)
      for dhi in range(D // 8):
        pltpu.async_copy(
            tbuf[b].at[pl.ds(dhi * 8 * BB, 8 * BB)],
            out_hbm.at[h, dhi, bb],
            wsem[b],
        )

    def wb_wait(g, b):
      h, bb = chunk_hb(g)
      for dhi in range(D // 8):
        pltpu.make_async_copy(
            tbuf[b].at[pl.ds(dhi * 8 * BB, 8 * BB)],
            out_hbm.at[h, dhi, bb],
            wsem[b],
        ).wait()

    dvec = lax.iota(jnp.int32, L)
    doff = [
        (jnp.int32(2 * q) + (dvec >> 3)) * (8 * BB) + (dvec & 7) * BB
        for q in range(D // L)
    ]

    WIN = 7 * (8 * BB) + 7 * (8 * BB) + 7 * BB + 8
    doff2 = [[dq + jnp.int32(j) for dq in doff] for j in range(8)]

    def transpose_scale(b):
      def body(r, c):
        base = r * ROWS_PER_ITER
        for i8 in range(ROWS_PER_ITER // 8):
          win = tbuf[b].at[pl.ds(base + i8 * 8, WIN)]
          for j in range(8):
            row = base + i8 * 8 + j
            for q in range(D // L):
              v = rows[b][row, pl.ds(q * L, L)] * SCALE
              plsc.store_scatter(win, [doff2[j][q]], v)
        return c

      lax.fori_loop(0, BB // ROWS_PER_ITER, body, 0)

    idx_sync(0, 0)
    idx_sync(1, 1)
    idx_sync(2, 2)
    gather_fire(0)
    gather_fire(1)

    def step(g, k, *, skip_isem_wait=False, fire_idx=True, refill=True,
             wait_wb=True):
      gather_wait(k)
      if wait_wb:
        wb_wait(g - NBUF, k)
      transpose_scale(k)
      wb_fire(g, k)
      if refill:
        b2 = (k + 2) % NBUF
        if not skip_isem_wait:
          idx_wait(g + 2, b2)
        gather_fire(b2)
        if fire_idx:
          idx_fire(g + 3, (k + 3) % NBUF)

    step(0, 0, skip_isem_wait=True, wait_wb=False)
    step(1, 1, wait_wb=False)
    step(2, 2, wait_wb=False)
    step(3, 3, wait_wb=False)

    def group(t, c):
      for k in range(NBUF):
        step(t * NBUF + k, k)
      return c

    lax.fori_loop(1, NT - 1, group, 0)

    g0 = NB - NBUF
    step(g0 + 0, 0)
    step(g0 + 1, 1, fire_idx=False)
    step(g0 + 2, 2, refill=False)
    step(g0 + 3, 3, refill=False)

    for k in range(NBUF):
      wb_wait(g0 + k, k)

  return sc_kernel


def kernel(x, table):
  out4 = _make_sc_kernel()(x.T.astype(jnp.int32), table)
  out5 = out4.reshape(HIST, D // 8, NBB, 8, BB)
  return out5.transpose(2, 4, 0, 1, 3).reshape(BATCH, HIST, D)


# speedup vs baseline: 1.9172x; 1.9048x over previous
import functools
import math

import jax
import jax.numpy as jnp
from jax import lax
from jax.experimental import pallas as pl
from jax.experimental.pallas import tpu as pltpu
from jax.experimental.pallas import tpu_sc as plsc

VOCAB = 1000000
D = 64
BATCH = 16384
HIST = 200

NC, NS, L = 2, 16, 16
NW = NC * NS

BB = 128
NBB = BATCH // BB
NCHUNK = HIST * NBB
NB = NCHUNK // NW
NBUF = 4
NT = NB // NBUF

SCALE = math.sqrt(D)

ROWS_PER_ITER = 16


def _make_sc_kernel():
  mesh = plsc.VectorSubcoreMesh(
      core_axis_name="c", subcore_axis_name="s", num_cores=NC
  )

  scratch = (
      [pltpu.VMEM((NBUF, BB), jnp.int32)]
      + [pltpu.VMEM((BB, D), jnp.float32)] * NBUF
      + [pltpu.VMEM((D, BB + 1), jnp.float32)] * NBUF
      + [pltpu.SemaphoreType.DMA] * (3 * NBUF)
  )

  @functools.partial(
      pl.kernel,
      mesh=mesh,
      out_type=jax.ShapeDtypeStruct((HIST, D // 8, NBB, 8, BB), jnp.float32),
      compiler_params=pltpu.CompilerParams(
          use_tc_tiling_on_sc=False, needs_layout_passes=False
      ),
      scratch_types=scratch,
  )
  def sc_kernel(idx_hbm, table_hbm, out_hbm, idx4, *rest):
    rows = rest[0:NBUF]
    tbuf = rest[NBUF : 2 * NBUF]
    sems = rest[2 * NBUF :]
    gsem = sems[0:NBUF]
    wsem = sems[NBUF : 2 * NBUF]
    isem = sems[2 * NBUF : 3 * NBUF]

    wid = lax.axis_index("s") * NC + lax.axis_index("c")
    wblk = wid * NB

    def chunk_hb(g):
      c = wblk + g
      return c // NBB, c % NBB

    def idx_sync(g, b):
      h, bb = chunk_hb(g)
      pltpu.sync_copy(idx_hbm.at[h, pl.ds(bb * BB, BB)], idx4.at[b])

    def idx_fire(g, b):
      h, bb = chunk_hb(g)
      pltpu.async_copy(idx_hbm.at[h, pl.ds(bb * BB, BB)], idx4.at[b], isem[b])

    def idx_wait(g, b):
      h, bb = chunk_hb(g)
      pltpu.make_async_copy(
          idx_hbm.at[h, pl.ds(bb * BB, BB)], idx4.at[b], isem[b]
      ).wait()

    def gather_fire(b):
      pltpu.async_copy(table_hbm.at[idx4.at[b]], rows[b], gsem[b])

    def gather_wait(b):
      pltpu.make_async_copy(table_hbm.at[idx4.at[b]], rows[b], gsem[b]).wait()

    def wb_fire(g, b):
      h, bb = chunk_hb(g)
      for dhi in range(D // 8):
        pltpu.async_copy(
            tbuf[b].at[pl.ds(dhi * 8, 8), pl.ds(0, BB)],
            out_hbm.at[h, dhi, bb],
            wsem[b],
        )

    def wb_wait(g, b):
      h, bb = chunk_hb(g)
      for dhi in range(D // 8):
        pltpu.make_async_copy(
            tbuf[b].at[pl.ds(dhi * 8, 8), pl.ds(0, BB)],
            out_hbm.at[h, dhi, bb],
            wsem[b],
        ).wait()

    dvec = lax.iota(jnp.int32, L)
    dconst = [jnp.int32(16 * q) + dvec for q in range(D // L)]

    def transpose_scale(b):
      def body(r, c):
        base = r * ROWS_PER_ITER
        for i in range(ROWS_PER_ITER):
          row = base + i
          rvec = jnp.full((L,), row, jnp.int32)
          for q in range(D // L):
            v = rows[b][row, pl.ds(q * L, L)] * SCALE
            plsc.store_scatter(tbuf[b], [dconst[q], rvec], v)
        return c

      lax.fori_loop(0, BB // ROWS_PER_ITER, body, 0)

    idx_sync(0, 0)
    idx_sync(1, 1)
    idx_sync(2, 2)
    gather_fire(0)
    gather_fire(1)

    def step(g, k, *, skip_isem_wait=False, fire_idx=True, refill=True,
             wait_wb=True):
      gather_wait(k)
      if wait_wb:
        wb_wait(g - NBUF, k)
      transpose_scale(k)
      wb_fire(g, k)
      if refill:
        b2 = (k + 2) % NBUF
        if not skip_isem_wait:
          idx_wait(g + 2, b2)
        gather_fire(b2)
        if fire_idx:
          idx_fire(g + 3, (k + 3) % NBUF)

    step(0, 0, skip_isem_wait=True, wait_wb=False)
    step(1, 1, wait_wb=False)
    step(2, 2, wait_wb=False)
    step(3, 3, wait_wb=False)

    def group(t, c):
      for k in range(NBUF):
        step(t * NBUF + k, k)
      return c

    lax.fori_loop(1, NT - 1, group, 0)

    g0 = NB - NBUF
    step(g0 + 0, 0)
    step(g0 + 1, 1, fire_idx=False)
    step(g0 + 2, 2, refill=False)
    step(g0 + 3, 3, refill=False)

    for k in range(NBUF):
      wb_wait(g0 + k, k)

  return sc_kernel


def kernel(x, table):
  out5 = _make_sc_kernel()(x.T.astype(jnp.int32), table)
  return out5.transpose(2, 4, 0, 1, 3).reshape(BATCH, HIST, D)


# speedup vs baseline: 3.4572x; 1.8033x over previous
import functools
import math

import jax
import jax.numpy as jnp
from jax import lax
from jax.experimental import pallas as pl
from jax.experimental.pallas import tpu as pltpu
from jax.experimental.pallas import tpu_sc as plsc

VOCAB = 1000000
D = 64
BATCH = 16384
HIST = 200

NC, NS, L = 2, 16, 16
NW = NC * NS

BB = 128
NBB = BATCH // BB
NCHUNK = HIST * NBB
NB = NCHUNK // NW
NBUF = 4
NT = NB // NBUF

SCALE = math.sqrt(D)

ROWS_PER_ITER = 16


def _make_sc_kernel():
  mesh = plsc.VectorSubcoreMesh(
      core_axis_name="c", subcore_axis_name="s", num_cores=NC
  )

  scratch = (
      [pltpu.VMEM((NBUF, BB), jnp.int32)]
      + [pltpu.VMEM((BB, D), jnp.float32)] * NBUF
      + [pltpu.VMEM((D, BB + 1), jnp.float32)] * NBUF
      + [pltpu.SemaphoreType.DMA] * (3 * NBUF)
  )

  @functools.partial(
      pl.kernel,
      mesh=mesh,
      out_type=jax.ShapeDtypeStruct((HIST, D // 8, NBB, 8, BB), jnp.float32),
      compiler_params=pltpu.CompilerParams(
          use_tc_tiling_on_sc=False, needs_layout_passes=False
      ),
      scratch_types=scratch,
  )
  def sc_kernel(idx_hbm, table_hbm, out_hbm, idx4, *rest):
    rows = rest[0:NBUF]
    tbuf = rest[NBUF : 2 * NBUF]
    sems = rest[2 * NBUF :]
    gsem = sems[0:NBUF]
    wsem = sems[NBUF : 2 * NBUF]
    isem = sems[2 * NBUF : 3 * NBUF]

    wid = lax.axis_index("s") * NC + lax.axis_index("c")
    wblk = wid * NB

    def chunk_hb(g):
      c = wblk + g
      return c // NBB, c % NBB

    def idx_sync(g, b):
      h, bb = chunk_hb(g)
      pltpu.sync_copy(idx_hbm.at[h, pl.ds(bb * BB, BB)], idx4.at[b])

    def idx_fire(g, b):
      h, bb = chunk_hb(g)
      pltpu.async_copy(idx_hbm.at[h, pl.ds(bb * BB, BB)], idx4.at[b], isem[b])

    def idx_wait(g, b):
      h, bb = chunk_hb(g)
      pltpu.make_async_copy(
          idx_hbm.at[h, pl.ds(bb * BB, BB)], idx4.at[b], isem[b]
      ).wait()

    def gather_fire(b):
      pltpu.async_copy(table_hbm.at[idx4.at[b]], rows[b], gsem[b])

    def gather_wait(b):
      pltpu.make_async_copy(table_hbm.at[idx4.at[b]], rows[b], gsem[b]).wait()

    def wb_fire(g, b):
      h, bb = chunk_hb(g)
      for dhi in range(D // 8):
        pltpu.async_copy(
            tbuf[b].at[pl.ds(dhi * 8, 8), pl.ds(0, BB)],
            out_hbm.at[h, dhi, bb],
            wsem[b],
        )

    def wb_wait(g, b):
      h, bb = chunk_hb(g)
      for dhi in range(D // 8):
        pltpu.make_async_copy(
            tbuf[b].at[pl.ds(dhi * 8, 8), pl.ds(0, BB)],
            out_hbm.at[h, dhi, bb],
            wsem[b],
        ).wait()

    dvec = lax.iota(jnp.int32, L)
    dconst = [jnp.int32(16 * q) + dvec for q in range(D // L)]

    def transpose_scale(b):
      @plsc.parallel_loop(0, BB, 1, unroll=ROWS_PER_ITER)
      def body(row):
        rvec = jnp.full((L,), row, jnp.int32)
        for q in range(D // L):
          v = rows[b][row, pl.ds(q * L, L)] * SCALE
          plsc.store_scatter(tbuf[b], [dconst[q], rvec], v)

    idx_sync(0, 0)
    idx_sync(1, 1)
    idx_sync(2, 2)
    gather_fire(0)
    gather_fire(1)

    def step(g, k, *, skip_isem_wait=False, fire_idx=True, refill=True,
             wait_wb=True):
      gather_wait(k)
      if wait_wb:
        wb_wait(g - NBUF, k)
      transpose_scale(k)
      wb_fire(g, k)
      if refill:
        b2 = (k + 2) % NBUF
        if not skip_isem_wait:
          idx_wait(g + 2, b2)
        gather_fire(b2)
        if fire_idx:
          idx_fire(g + 3, (k + 3) % NBUF)

    step(0, 0, skip_isem_wait=True, wait_wb=False)
    step(1, 1, wait_wb=False)
    step(2, 2, wait_wb=False)
    step(3, 3, wait_wb=False)

    def group(t, c):
      for k in range(NBUF):
        step(t * NBUF + k, k)
      return c

    lax.fori_loop(1, NT - 1, group, 0)

    g0 = NB - NBUF
    step(g0 + 0, 0)
    step(g0 + 1, 1, fire_idx=False)
    step(g0 + 2, 2, refill=False)
    step(g0 + 3, 3, refill=False)

    for k in range(NBUF):
      wb_wait(g0 + k, k)

  return sc_kernel


def kernel(x, table):
  out5 = _make_sc_kernel()(x.T.astype(jnp.int32), table)
  return out5.transpose(2, 4, 0, 1, 3).reshape(BATCH, HIST, D)
